# Initial kernel scaffold; baseline (speedup 1.0000x reference)
import jax
import jax.numpy as jnp
from jax.experimental import pallas as pl


def kernel(feat, edge_index, W, b):
    raise NotImplementedError("write your pallas kernel here")


# speedup vs baseline: 8.1392x; 8.1392x over previous
import functools

import jax
import jax.numpy as jnp
from jax import lax
from jax.experimental import pallas as pl
from jax.experimental.pallas import tpu as pltpu
from jax.experimental.pallas import tpu_sc as plsc

N = 10000
E = 320000
D = 128
NP = 10240

NC = 2
NS = 16
NW = NC * NS
EPW = E // NW
C = 80
NCH = EPW // C
RPS = NP // NS
RB = 1024

_mesh = plsc.VectorSubcoreMesh(core_axis_name="c", subcore_axis_name="s")


@functools.partial(
    pl.kernel,
    out_type=jax.ShapeDtypeStruct((NC, 2, NP), jnp.float32),
    mesh=_mesh,
    scratch_types=[
        pltpu.VMEM((NCH, C), jnp.int32),
        pltpu.VMEM((C,), jnp.float32),
        pltpu.VMEM_SHARED((NP,), jnp.float32),
        pltpu.VMEM_SHARED((NP,), jnp.float32),
    ],
)
def _sc_degrees(src_hbm, dst_hbm, zeros_hbm, ones_hbm, out_hbm,
                idx_v, ones_v, degs_sh, degd_sh):
    cid = lax.axis_index("c")
    sid = lax.axis_index("s")
    wid = cid * NS + sid

    pltpu.sync_copy(ones_hbm, ones_v)
    sl = pl.ds(sid * RPS, RPS)
    pltpu.sync_copy(zeros_hbm.at[sl], degs_sh.at[sl])
    pltpu.sync_copy(zeros_hbm.at[sl], degd_sh.at[sl])
    plsc.subcore_barrier()

    pltpu.sync_copy(src_hbm.at[wid], idx_v)

    @pl.loop(0, NCH)
    def _(j):
        pltpu.sync_copy(ones_v, degs_sh.at[idx_v.at[j]], add=True)

    pltpu.sync_copy(dst_hbm.at[wid], idx_v)

    @pl.loop(0, NCH)
    def _(j):
        pltpu.sync_copy(ones_v, degd_sh.at[idx_v.at[j]], add=True)

    plsc.subcore_barrier()
    pltpu.sync_copy(degs_sh.at[sl], out_hbm.at[cid, 0, sl])
    pltpu.sync_copy(degd_sh.at[sl], out_hbm.at[cid, 1, sl])


@functools.partial(
    pl.kernel,
    out_type=jax.ShapeDtypeStruct((NC, NP, D), jnp.float32),
    mesh=_mesh,
    scratch_types=[
        pltpu.VMEM((NCH, C), jnp.int32),
        pltpu.VMEM((NCH, C), jnp.int32),
        pltpu.VMEM((C, D), jnp.float32),
        pltpu.VMEM_SHARED((NP, D), jnp.float32),
    ],
)
def _sc_aggregate(h_hbm, src_hbm, dst_hbm, zeros_hbm, out_hbm,
                  src_v, dst_v, rows_v, agg_sh):
    cid = lax.axis_index("c")
    sid = lax.axis_index("s")
    wid = cid * NS + sid

    sl = pl.ds(sid * RPS, RPS)
    pltpu.sync_copy(zeros_hbm.at[sl], agg_sh.at[sl])
    pltpu.sync_copy(src_hbm.at[wid], src_v)
    pltpu.sync_copy(dst_hbm.at[wid], dst_v)
    plsc.subcore_barrier()

    @pl.loop(0, NCH)
    def _(j):
        pltpu.sync_copy(h_hbm.at[src_v.at[j]], rows_v)
        pltpu.sync_copy(rows_v, agg_sh.at[dst_v.at[j]], add=True)

    plsc.subcore_barrier()
    pltpu.sync_copy(agg_sh.at[sl], out_hbm.at[cid, sl])


def _tc_pre_body(feat_ref, w_ref, degp_ref, h_ref):
    d = degp_ref[...]
    deg = d[0, 0, 0, :] + d[1, 0, 0, :]
    norm = lax.rsqrt(jnp.maximum(deg, 1.0))
    t = jnp.dot(feat_ref[...], w_ref[...], preferred_element_type=jnp.float32)
    h_ref[...] = t * norm[:, None]


_tc_pre = pl.pallas_call(
    _tc_pre_body,
    grid=(NP // RB,),
    in_specs=[
        pl.BlockSpec((RB, D), lambda i: (i, 0)),
        pl.BlockSpec((D, D), lambda i: (0, 0)),
        pl.BlockSpec((NC, 1, 1, RB), lambda i: (0, i, 0, 0)),
    ],
    out_specs=pl.BlockSpec((RB, D), lambda i: (i, 0)),
    out_shape=jax.ShapeDtypeStruct((NP, D), jnp.float32),
)


def _tc_post_body(parts_ref, degp_ref, b_ref, out_ref):
    p = parts_ref[...]
    d = degp_ref[...]
    deg = d[0, 0, 0, :] + d[1, 0, 0, :]
    norm = lax.rsqrt(jnp.maximum(deg, 1.0))
    agg = (p[0] + p[1]) * norm[:, None]
    out_ref[...] = jnp.maximum(agg + b_ref[...], 0.0)


_tc_post = pl.pallas_call(
    _tc_post_body,
    grid=(NP // RB,),
    in_specs=[
        pl.BlockSpec((NC, RB, D), lambda i: (0, i, 0)),
        pl.BlockSpec((NC, 1, 1, RB), lambda i: (0, i, 0, 0)),
        pl.BlockSpec((1, D), lambda i: (0, 0)),
    ],
    out_specs=pl.BlockSpec((RB, D), lambda i: (i, 0)),
    out_shape=jax.ShapeDtypeStruct((NP, D), jnp.float32),
)


def kernel(feat, edge_index, W, b):
    src = edge_index[0].reshape(NW, NCH, C)
    dst = edge_index[1].reshape(NW, NCH, C)
    zeros1 = jnp.zeros((NP,), jnp.float32)
    ones_c = jnp.ones((C,), jnp.float32)
    zeros2 = jnp.zeros((NP, D), jnp.float32)
    feat_p = jnp.pad(feat, ((0, NP - N), (0, 0)))

    degp = _sc_degrees(src, dst, zeros1, ones_c)
    deg_out = degp[:, 0, :].reshape(NC, NP // RB, 1, RB)
    deg_in = degp[:, 1, :].reshape(NC, NP // RB, 1, RB)

    h = _tc_pre(feat_p, W, deg_out)
    parts = _sc_aggregate(h, src, dst, zeros2)
    out = _tc_post(parts, deg_in, b.reshape(1, D))
    return out[:N]


# speedup vs baseline: 11.1237x; 1.3667x over previous
import functools

import jax
import jax.numpy as jnp
from jax import lax
from jax.experimental import pallas as pl
from jax.experimental.pallas import tpu as pltpu
from jax.experimental.pallas import tpu_sc as plsc

N = 10000
E = 320000
D = 128
NP = 10240

NC = 2
NS = 16
NW = NC * NS
EPW = E // NW
C = 80
NCH = EPW // C
RPS = NP // NS
NRS = N // NS
RB = 1000

_mesh = plsc.VectorSubcoreMesh(core_axis_name="c", subcore_axis_name="s")


@functools.partial(
    pl.kernel,
    out_type=jax.ShapeDtypeStruct((NC, 2, NP), jnp.float32),
    mesh=_mesh,
    scratch_types=[
        pltpu.VMEM((NCH, 2, C), jnp.int32),
        pltpu.VMEM((C,), jnp.float32),
        pltpu.VMEM_SHARED((NP,), jnp.float32),
        pltpu.VMEM_SHARED((NP,), jnp.float32),
    ],
)
def _sc_degrees(edges_hbm, zeros_hbm, ones_hbm, out_hbm,
                idx_v, ones_v, degs_sh, degd_sh):
    cid = lax.axis_index("c")
    sid = lax.axis_index("s")
    wid = cid * NS + sid

    pltpu.sync_copy(ones_hbm, ones_v)
    sl = pl.ds(sid * RPS, RPS)
    pltpu.sync_copy(zeros_hbm.at[sl], degs_sh.at[sl])
    pltpu.sync_copy(zeros_hbm.at[sl], degd_sh.at[sl])
    pltpu.sync_copy(edges_hbm.at[wid], idx_v)
    plsc.subcore_barrier()

    @pl.loop(0, NCH)
    def _(j):
        pltpu.sync_copy(ones_v, degs_sh.at[idx_v.at[j, 0]], add=True)
        pltpu.sync_copy(ones_v, degd_sh.at[idx_v.at[j, 1]], add=True)

    plsc.subcore_barrier()
    pltpu.sync_copy(degs_sh.at[sl], out_hbm.at[cid, 0, sl])
    pltpu.sync_copy(degd_sh.at[sl], out_hbm.at[cid, 1, sl])


PH1 = 63
PH2 = NCH - PH1


def _run_phase(h_hbm, idx_v, rows_a, rows_b, agg_sh, sem_a, sem_b, n):
    pltpu.async_copy(h_hbm.at[idx_v.at[0, 0]], rows_a, sem_a)
    pltpu.async_copy(h_hbm.at[idx_v.at[1, 0]], rows_b, sem_b)

    end = n - 1 if n % 2 else n - 2

    @pl.loop(0, end, step=2)
    def _(j):
        pltpu.make_async_copy(h_hbm.at[idx_v.at[j, 0]], rows_a, sem_a).wait()
        pltpu.sync_copy(rows_a, agg_sh.at[idx_v.at[j, 1]], add=True)

        @pl.when(j + 2 < n)
        def _():
            pltpu.async_copy(h_hbm.at[idx_v.at[j + 2, 0]], rows_a, sem_a)

        pltpu.make_async_copy(h_hbm.at[idx_v.at[j + 1, 0]], rows_b, sem_b).wait()
        pltpu.sync_copy(rows_b, agg_sh.at[idx_v.at[j + 1, 1]], add=True)

        @pl.when(j + 3 < n)
        def _():
            pltpu.async_copy(h_hbm.at[idx_v.at[j + 3, 0]], rows_b, sem_b)

    if n % 2:
        pltpu.make_async_copy(h_hbm.at[idx_v.at[n - 1, 0]], rows_a, sem_a).wait()
        pltpu.sync_copy(rows_a, agg_sh.at[idx_v.at[n - 1, 1]], add=True)
    else:
        pltpu.make_async_copy(h_hbm.at[idx_v.at[n - 2, 0]], rows_a, sem_a).wait()
        pltpu.sync_copy(rows_a, agg_sh.at[idx_v.at[n - 2, 1]], add=True)
        pltpu.make_async_copy(h_hbm.at[idx_v.at[n - 1, 0]], rows_b, sem_b).wait()
        pltpu.sync_copy(rows_b, agg_sh.at[idx_v.at[n - 1, 1]], add=True)


@functools.partial(
    pl.kernel,
    out_type=jax.ShapeDtypeStruct((NC, N, D), jnp.float32),
    mesh=_mesh,
    scratch_types=[
        pltpu.VMEM((PH1, 2, C), jnp.int32),
        pltpu.VMEM((C, D), jnp.float32),
        pltpu.VMEM((C, D), jnp.float32),
        pltpu.VMEM_SHARED((N, D), jnp.float32),
        pltpu.SemaphoreType.DMA,
        pltpu.SemaphoreType.DMA,
    ],
)
def _sc_aggregate(h_hbm, edges_hbm, zeros_hbm, out_hbm,
                  idx_v, rows_a, rows_b, agg_sh, sem_a, sem_b):
    cid = lax.axis_index("c")
    sid = lax.axis_index("s")
    wid = cid * NS + sid

    @pl.when(sid < N // RB)
    def _():
        sl = pl.ds(pl.multiple_of(sid * RB, 8), RB)
        pltpu.sync_copy(zeros_hbm.at[sl], agg_sh.at[sl])

    pltpu.sync_copy(edges_hbm.at[wid, pl.ds(0, PH1)], idx_v)
    plsc.subcore_barrier()

    _run_phase(h_hbm, idx_v, rows_a, rows_b, agg_sh, sem_a, sem_b, PH1)
    pltpu.sync_copy(edges_hbm.at[wid, pl.ds(PH1, PH2)], idx_v.at[pl.ds(0, PH2)])
    _run_phase(h_hbm, idx_v, rows_a, rows_b, agg_sh, sem_a, sem_b, PH2)

    plsc.subcore_barrier()

    @pl.when(sid < N // RB)
    def _():
        sl = pl.ds(pl.multiple_of(sid * RB, 8), RB)
        pltpu.sync_copy(agg_sh.at[sl], out_hbm.at[cid, sl])


def _tc_pre_body(feat_ref, w_ref, degp_ref, h_ref):
    d = degp_ref[...]
    deg = d[0, 0, 0, :] + d[1, 0, 0, :]
    norm = lax.rsqrt(jnp.maximum(deg, 1.0))
    t = jnp.dot(feat_ref[...], w_ref[...], preferred_element_type=jnp.float32)
    h_ref[...] = t * norm[:, None]


_tc_pre = pl.pallas_call(
    _tc_pre_body,
    grid=(N // RB,),
    in_specs=[
        pl.BlockSpec((RB, D), lambda i: (i, 0)),
        pl.BlockSpec((D, D), lambda i: (0, 0)),
        pl.BlockSpec((NC, 1, 1, RB), lambda i: (0, i, 0, 0)),
    ],
    out_specs=pl.BlockSpec((RB, D), lambda i: (i, 0)),
    out_shape=jax.ShapeDtypeStruct((N, D), jnp.float32),
)


def _tc_post_body(parts_ref, degp_ref, b_ref, out_ref):
    p = parts_ref[...]
    d = degp_ref[...]
    deg = d[0, 0, 0, :] + d[1, 0, 0, :]
    norm = lax.rsqrt(jnp.maximum(deg, 1.0))
    agg = (p[0] + p[1]) * norm[:, None]
    out_ref[...] = jnp.maximum(agg + b_ref[...], 0.0)


_tc_post = pl.pallas_call(
    _tc_post_body,
    grid=(N // RB,),
    in_specs=[
        pl.BlockSpec((NC, RB, D), lambda i: (0, i, 0)),
        pl.BlockSpec((NC, 1, 1, RB), lambda i: (0, i, 0, 0)),
        pl.BlockSpec((1, D), lambda i: (0, 0)),
    ],
    out_specs=pl.BlockSpec((RB, D), lambda i: (i, 0)),
    out_shape=jax.ShapeDtypeStruct((N, D), jnp.float32),
)


def kernel(feat, edge_index, W, b):
    edges = jnp.stack(
        [edge_index[0].reshape(NW, NCH, C), edge_index[1].reshape(NW, NCH, C)],
        axis=2,
    )
    zeros1 = jnp.zeros((NP,), jnp.float32)
    ones_c = jnp.ones((C,), jnp.float32)
    zeros2 = jnp.zeros((N, D), jnp.float32)

    degp = _sc_degrees(edges, zeros1, ones_c)
    deg_out = degp[:, 0, :N].reshape(NC, N // RB, 1, RB)
    deg_in = degp[:, 1, :N].reshape(NC, N // RB, 1, RB)

    h = _tc_pre(feat, W, deg_out)
    parts = _sc_aggregate(h, edges, zeros2)
    return _tc_post(parts, deg_in, b.reshape(1, D))
